# baseline (speedup 1.0000x reference)
import functools

import jax
import jax.numpy as jnp
import numpy as np
from jax import lax
from jax.experimental import pallas as pl
from jax.experimental.pallas import tpu as pltpu
from jax.experimental.pallas import tpu_sc as plsc

_NUM_NODES = 100000
_D = 128
_CW = 2
_B = 4096
_S = 50
_NC = _S - 2 * _CW - 2
_K = 2 * _CW
_NP = _NC * _K
_L = 16
_G = _D // _L
_NW = 32
_RPW = _B // _NW
_HALF = _NP // 2
_OFFS = (0, 1, 3, 4)


def _sc_scores(seq, W, rnd):
    mesh = plsc.VectorSubcoreMesh(core_axis_name="c", subcore_axis_name="s")

    @functools.partial(
        pl.kernel,
        out_type=(
            jax.ShapeDtypeStruct((_B, _NP), jnp.float32),
            jax.ShapeDtypeStruct((_B, _NP), jnp.float32),
        ),
        mesh=mesh,
        compiler_params=pltpu.CompilerParams(needs_layout_passes=False),
        scratch_types=[
            pltpu.VMEM((_RPW, _S), jnp.int32),
            pltpu.VMEM((_RPW, 2, _HALF), jnp.int32),
            pltpu.VMEM((2 * _S, _D), jnp.float32),
            pltpu.VMEM((2 * _NP, _D), jnp.float32),
            pltpu.VMEM((_NP * _L,), jnp.float32),
            pltpu.VMEM((_NP * _L,), jnp.float32),
            pltpu.VMEM((_NP,), jnp.float32),
            pltpu.VMEM((_NP,), jnp.float32),
            pltpu.VMEM((_NP,), jnp.float32),
            pltpu.VMEM((_NP,), jnp.float32),
            pltpu.SemaphoreType.DMA,
            pltpu.SemaphoreType.DMA,
            pltpu.SemaphoreType.DMA,
            pltpu.SemaphoreType.DMA,
        ],
    )
    def k(seq_hbm, w_hbm, rnd_hbm, pos_hbm, neg_hbm,
          seq_v, rnd_v, e_v, r_v, pacc, nacc, pos_s0, pos_s1, neg_s0, neg_s1,
          gsem0, gsem1, osem0, osem1):
        pos_sb = (pos_s0, pos_s1)
        neg_sb = (neg_s0, neg_s1)
        wid = lax.axis_index("s") * 2 + lax.axis_index("c")
        base = wid * _RPW
        pltpu.sync_copy(seq_hbm.at[pl.ds(base, _RPW)], seq_v)
        pltpu.sync_copy(rnd_hbm.at[pl.ds(base, _RPW)], rnd_v)
        lane = lax.iota(jnp.int32, _L)
        gsems = (gsem0, gsem1)
        osems = (osem0, osem1)

        def gather_copies(r, q, sem):
            return (
                pltpu.make_async_copy(w_hbm.at[seq_v.at[r]],
                                      e_v.at[pl.ds(q * _S, _S)], sem),
                pltpu.make_async_copy(w_hbm.at[rnd_v.at[r, 0]],
                                      r_v.at[pl.ds(q * _NP, _HALF)], sem),
                pltpu.make_async_copy(w_hbm.at[rnd_v.at[r, 1]],
                                      r_v.at[pl.ds(q * _NP + _HALF, _HALF)], sem),
            )

        def issue_gathers(r, q):
            for c in gather_copies(r, q, gsems[q]):
                c.start()

        def drain_gathers(r, q):
            for c in gather_copies(r, q, gsems[q]):
                c.wait()

        issue_gathers(0, 0)
        issue_gathers(1, 1)

        def pair_body(g, carry):
            for q in (0, 1):
                r = 2 * g + q
                b = base + r
                drain_gathers(r, q)

                def erow(idx):
                    return [e_v[idx, pl.ds(gg * _L, _L)] for gg in range(_G)]

                def dot(a, b):
                    acc = a[0] * b[0]
                    for gg in range(1, _G):
                        acc = acc + a[gg] * b[gg]
                    return acc

                def rdot(a, jq):
                    acc = a[0] * r_v[jq, pl.ds(0, _L)]
                    for gg in range(1, _G):
                        acc = acc + a[gg] * r_v[jq, pl.ds(gg * _L, _L)]
                    return acc

                @plsc.parallel_loop(0, _NC, 2, unroll=1)
                def _(i):
                    iq = i + q * _S
                    c0 = erow(iq + 2)
                    c1 = erow(iq + 3)
                    s1 = erow(iq + 1)
                    s4 = erow(iq + 4)
                    d01 = dot(c0, c1)
                    j0 = i * _K
                    pacc[pl.ds((j0 + 0) * _L, _L)] = dot(c0, erow(iq))
                    pacc[pl.ds((j0 + 1) * _L, _L)] = dot(c0, s1)
                    pacc[pl.ds((j0 + 2) * _L, _L)] = d01
                    pacc[pl.ds((j0 + 3) * _L, _L)] = dot(c0, s4)
                    pacc[pl.ds((j0 + 4) * _L, _L)] = dot(c1, s1)
                    pacc[pl.ds((j0 + 5) * _L, _L)] = d01
                    pacc[pl.ds((j0 + 6) * _L, _L)] = dot(c1, s4)
                    pacc[pl.ds((j0 + 7) * _L, _L)] = dot(c1, erow(iq + 5))
                    for z in range(_K):
                        nacc[pl.ds((j0 + z) * _L, _L)] = rdot(
                            c0, j0 + z + q * _NP)
                        nacc[pl.ds((j0 + _K + z) * _L, _L)] = rdot(
                            c1, j0 + _K + z + q * _NP)

                @pl.when(g < _RPW // 2 - 1)
                def _():
                    issue_gathers(r + 2, q)

                @pl.when(g >= 1)
                def _():
                    pltpu.make_async_copy(pos_sb[q], pos_hbm.at[b], osems[q]).wait()
                    pltpu.make_async_copy(neg_sb[q], neg_hbm.at[b], osems[q]).wait()

                @plsc.parallel_loop(0, _NP // _L, 1, unroll=2)
                def _(chunk):
                    rows = (chunk * _L + lane) * _L
                    tp = plsc.load_gather(pacc, [rows])
                    tn = plsc.load_gather(nacc, [rows])
                    for l in range(1, _L):
                        tp = tp + plsc.load_gather(pacc, [rows + l])
                        tn = tn + plsc.load_gather(nacc, [rows + l])
                    pos_sb[q][pl.ds(chunk * _L, _L)] = tp
                    neg_sb[q][pl.ds(chunk * _L, _L)] = tn

                pltpu.async_copy(pos_sb[q], pos_hbm.at[b], osems[q])
                pltpu.async_copy(neg_sb[q], neg_hbm.at[b], osems[q])
            return carry

        lax.fori_loop(0, _RPW // 2, pair_body, 0)

        for q in (0, 1):
            pltpu.make_async_copy(pos_sb[q], pos_hbm.at[base], osems[q]).wait()
            pltpu.make_async_copy(neg_sb[q], neg_hbm.at[base], osems[q]).wait()

    return k(seq, W, rnd)


def _loss_tc(pos, neg):
    def body(pos_ref, neg_ref, out_ref):
        p = pos_ref[...]
        n = neg_ref[...]
        lp = jnp.log(jax.nn.sigmoid(p) + 1e-08)
        ln = jnp.log(1.0 - jax.nn.sigmoid(n) + 1e-08)
        inv = 1.0 / float(_B * _NP)
        out_ref[0, 0] = -(jnp.sum(jnp.sum(lp, axis=1)) * inv
                          + jnp.sum(jnp.sum(ln, axis=1)) * inv)

    return pl.pallas_call(
        body,
        out_shape=jax.ShapeDtypeStruct((1, 1), jnp.float32),
        out_specs=pl.BlockSpec(memory_space=pltpu.SMEM),
    )(pos, neg)


_RND_CACHE = {}


def _rnd_ids():
    if "v" not in _RND_CACHE:
        with jax.ensure_compile_time_eval():
            r = jax.random.randint(jax.random.key(1), (_B, _NC, _K),
                                   0, _NUM_NODES)
            _RND_CACHE["v"] = np.asarray(r).reshape(_B, 2, _HALF).astype(np.int32)
    return _RND_CACHE["v"]


def kernel(seq, W):
    seq = seq.astype(jnp.int32)
    pos, neg = _sc_scores(seq, W, _rnd_ids())
    return _loss_tc(pos, neg)[0, 0]


# speedup vs baseline: 1.0295x; 1.0295x over previous
import functools

import jax
import jax.numpy as jnp
import numpy as np
from jax import lax
from jax.experimental import pallas as pl
from jax.experimental.pallas import tpu as pltpu
from jax.experimental.pallas import tpu_sc as plsc

_NUM_NODES = 100000
_D = 128
_CW = 2
_B = 4096
_S = 50
_NC = _S - 2 * _CW - 2
_K = 2 * _CW
_NP = _NC * _K
_L = 16
_G = _D // _L
_NW = 32
_RPW = _B // _NW
_HALF = _NP // 2
_OFFS = (0, 1, 3, 4)


def _sc_scores(seq, W, rnd):
    mesh = plsc.VectorSubcoreMesh(core_axis_name="c", subcore_axis_name="s")

    @functools.partial(
        pl.kernel,
        out_type=(
            jax.ShapeDtypeStruct((_B, _NP), jnp.float32),
            jax.ShapeDtypeStruct((_B, _NP), jnp.float32),
        ),
        mesh=mesh,
        compiler_params=pltpu.CompilerParams(needs_layout_passes=False),
        scratch_types=[
            pltpu.VMEM((_RPW, _S), jnp.int32),
            pltpu.VMEM((_RPW, 2, _HALF), jnp.int32),
            pltpu.VMEM((2 * _S, _D), jnp.float32),
            pltpu.VMEM((2 * _NP, _D), jnp.float32),
            pltpu.VMEM((_NP * _L,), jnp.float32),
            pltpu.VMEM((_NP * _L,), jnp.float32),
            pltpu.VMEM((_NP,), jnp.float32),
            pltpu.VMEM((_NP,), jnp.float32),
            pltpu.VMEM((_NP,), jnp.float32),
            pltpu.VMEM((_NP,), jnp.float32),
            pltpu.SemaphoreType.DMA,
            pltpu.SemaphoreType.DMA,
            pltpu.SemaphoreType.DMA,
            pltpu.SemaphoreType.DMA,
        ],
    )
    def k(seq_hbm, w_hbm, rnd_hbm, pos_hbm, neg_hbm,
          seq_v, rnd_v, e_v, r_v, pacc, nacc, pos_s0, pos_s1, neg_s0, neg_s1,
          gsem0, gsem1, osem0, osem1):
        pos_sb = (pos_s0, pos_s1)
        neg_sb = (neg_s0, neg_s1)
        wid = lax.axis_index("s") * 2 + lax.axis_index("c")
        base = wid * _RPW
        pltpu.sync_copy(seq_hbm.at[pl.ds(base, _RPW)], seq_v)
        pltpu.sync_copy(rnd_hbm.at[pl.ds(base, _RPW)], rnd_v)
        lane = lax.iota(jnp.int32, _L)
        gsems = (gsem0, gsem1)
        osems = (osem0, osem1)

        def gather_copies(r, q, sem):
            return (
                pltpu.make_async_copy(w_hbm.at[seq_v.at[r]],
                                      e_v.at[pl.ds(q * _S, _S)], sem),
                pltpu.make_async_copy(w_hbm.at[rnd_v.at[r, 0]],
                                      r_v.at[pl.ds(q * _NP, _HALF)], sem),
                pltpu.make_async_copy(w_hbm.at[rnd_v.at[r, 1]],
                                      r_v.at[pl.ds(q * _NP + _HALF, _HALF)], sem),
            )

        def issue_gathers(r, q):
            for c in gather_copies(r, q, gsems[q]):
                c.start()

        def drain_gathers(r, q):
            for c in gather_copies(r, q, gsems[q]):
                c.wait()

        issue_gathers(0, 0)
        issue_gathers(1, 1)

        def pair_body(g, carry):
            for q in (0, 1):
                r = 2 * g + q
                b = base + r
                drain_gathers(r, q)

                @plsc.parallel_loop(0, _NC, 1, unroll=2)
                def _(i):
                    m = i + _CW + q * _S
                    cvec = [e_v[m, pl.ds(gg * _L, _L)] for gg in range(_G)]
                    for kk, off in enumerate(_OFFS):
                        p = i + off + q * _S
                        j = i * _K + kk
                        jq = j + q * _NP
                        acc = cvec[0] * e_v[p, pl.ds(0, _L)]
                        for gg in range(1, _G):
                            acc = acc + cvec[gg] * e_v[p, pl.ds(gg * _L, _L)]
                        pacc[pl.ds(j * _L, _L)] = acc
                        acc = cvec[0] * r_v[jq, pl.ds(0, _L)]
                        for gg in range(1, _G):
                            acc = acc + cvec[gg] * r_v[jq, pl.ds(gg * _L, _L)]
                        nacc[pl.ds(j * _L, _L)] = acc

                @pl.when(g < _RPW // 2 - 1)
                def _():
                    issue_gathers(r + 2, q)

                @pl.when(g >= 1)
                def _():
                    pltpu.make_async_copy(pos_sb[q], pos_hbm.at[b], osems[q]).wait()
                    pltpu.make_async_copy(neg_sb[q], neg_hbm.at[b], osems[q]).wait()

                @plsc.parallel_loop(0, _NP // _L, 1, unroll=3)
                def _(chunk):
                    rows = (chunk * _L + lane) * _L
                    tp = plsc.load_gather(pacc, [rows])
                    tn = plsc.load_gather(nacc, [rows])
                    for l in range(1, _L):
                        tp = tp + plsc.load_gather(pacc, [rows + l])
                        tn = tn + plsc.load_gather(nacc, [rows + l])
                    pos_sb[q][pl.ds(chunk * _L, _L)] = tp
                    neg_sb[q][pl.ds(chunk * _L, _L)] = tn

                pltpu.async_copy(pos_sb[q], pos_hbm.at[b], osems[q])
                pltpu.async_copy(neg_sb[q], neg_hbm.at[b], osems[q])
            return carry

        lax.fori_loop(0, _RPW // 2, pair_body, 0)

        for q in (0, 1):
            pltpu.make_async_copy(pos_sb[q], pos_hbm.at[base], osems[q]).wait()
            pltpu.make_async_copy(neg_sb[q], neg_hbm.at[base], osems[q]).wait()

    return k(seq, W, rnd)


def _loss_tc(pos, neg):
    def body(pos_ref, neg_ref, out_ref):
        p = pos_ref[...]
        n = neg_ref[...]
        lp = jnp.log(jax.nn.sigmoid(p) + 1e-08)
        ln = jnp.log(1.0 - jax.nn.sigmoid(n) + 1e-08)
        inv = 1.0 / float(_B * _NP)
        out_ref[0, 0] = -(jnp.sum(jnp.sum(lp, axis=1)) * inv
                          + jnp.sum(jnp.sum(ln, axis=1)) * inv)

    return pl.pallas_call(
        body,
        out_shape=jax.ShapeDtypeStruct((1, 1), jnp.float32),
        out_specs=pl.BlockSpec(memory_space=pltpu.SMEM),
    )(pos, neg)


_RND_CACHE = {}


def _rnd_ids():
    if "v" not in _RND_CACHE:
        with jax.ensure_compile_time_eval():
            r = jax.random.randint(jax.random.key(1), (_B, _NC, _K),
                                   0, _NUM_NODES)
            _RND_CACHE["v"] = np.asarray(r).reshape(_B, 2, _HALF).astype(np.int32)
    return _RND_CACHE["v"]


def kernel(seq, W):
    seq = seq.astype(jnp.int32)
    pos, neg = _sc_scores(seq, W, _rnd_ids())
    return _loss_tc(pos, neg)[0, 0]


# speedup vs baseline: 1.0703x; 1.0396x over previous
import functools

import jax
import jax.numpy as jnp
import numpy as np
from jax import lax
from jax.experimental import pallas as pl
from jax.experimental.pallas import tpu as pltpu
from jax.experimental.pallas import tpu_sc as plsc

_NUM_NODES = 100000
_D = 128
_CW = 2
_B = 4096
_S = 50
_NC = _S - 2 * _CW - 2
_K = 2 * _CW
_NP = _NC * _K
_L = 16
_G = _D // _L
_NW = 32
_RPW = _B // _NW
_HALF = _NP // 2
_OFFS = (0, 1, 3, 4)


def _sc_scores(seq, W, rnd):
    mesh = plsc.VectorSubcoreMesh(core_axis_name="c", subcore_axis_name="s")

    @functools.partial(
        pl.kernel,
        out_type=(
            jax.ShapeDtypeStruct((_B, _NP), jnp.float32),
            jax.ShapeDtypeStruct((_B, _NP), jnp.float32),
        ),
        mesh=mesh,
        compiler_params=pltpu.CompilerParams(needs_layout_passes=False),
        scratch_types=[
            pltpu.VMEM((_RPW, _S), jnp.int32),
            pltpu.VMEM((_RPW, 2, _HALF), jnp.int32),
            pltpu.VMEM((2 * _S, _D), jnp.float32),
            pltpu.VMEM((2 * _NP, _D), jnp.float32),
            pltpu.VMEM((_NP * _L,), jnp.float32),
            pltpu.VMEM((_NP * _L,), jnp.float32),
            pltpu.VMEM((_NP,), jnp.float32),
            pltpu.VMEM((_NP,), jnp.float32),
            pltpu.VMEM((_NP,), jnp.float32),
            pltpu.VMEM((_NP,), jnp.float32),
            pltpu.SemaphoreType.DMA,
            pltpu.SemaphoreType.DMA,
            pltpu.SemaphoreType.DMA,
            pltpu.SemaphoreType.DMA,
        ],
    )
    def k(seq_hbm, w_hbm, rnd_hbm, pos_hbm, neg_hbm,
          seq_v, rnd_v, e_v, r_v, pacc, nacc, pos_s0, pos_s1, neg_s0, neg_s1,
          gsem0, gsem1, osem0, osem1):
        pos_sb = (pos_s0, pos_s1)
        neg_sb = (neg_s0, neg_s1)
        wid = lax.axis_index("s") * 2 + lax.axis_index("c")
        base = wid * _RPW
        pltpu.sync_copy(seq_hbm.at[pl.ds(base, _RPW)], seq_v)
        pltpu.sync_copy(rnd_hbm.at[pl.ds(base, _RPW)], rnd_v)
        lane = lax.iota(jnp.int32, _L)
        gsems = (gsem0, gsem1)
        osems = (osem0, osem1)

        def gather_copies(r, q, sem):
            return (
                pltpu.make_async_copy(w_hbm.at[seq_v.at[r]],
                                      e_v.at[pl.ds(q * _S, _S)], sem),
                pltpu.make_async_copy(w_hbm.at[rnd_v.at[r, 0]],
                                      r_v.at[pl.ds(q * _NP, _HALF)], sem),
                pltpu.make_async_copy(w_hbm.at[rnd_v.at[r, 1]],
                                      r_v.at[pl.ds(q * _NP + _HALF, _HALF)], sem),
            )

        def issue_gathers(r, q):
            for c in gather_copies(r, q, gsems[q]):
                c.start()

        def drain_gathers(r, q):
            for c in gather_copies(r, q, gsems[q]):
                c.wait()

        issue_gathers(0, 0)
        issue_gathers(1, 1)

        def pair_body(g, carry):
            for q in (0, 1):
                r = 2 * g + q
                b = base + r
                drain_gathers(r, q)

                @plsc.parallel_loop(0, _NC, 1, unroll=2)
                def _(i):
                    m = i + _CW + q * _S
                    cvec = [e_v[m, pl.ds(gg * _L, _L)] for gg in range(_G)]
                    for kk, off in enumerate(_OFFS):
                        p = i + off + q * _S
                        j = i * _K + kk
                        jq = j + q * _NP
                        acc = cvec[0] * e_v[p, pl.ds(0, _L)]
                        for gg in range(1, _G):
                            acc = acc + cvec[gg] * e_v[p, pl.ds(gg * _L, _L)]
                        pacc[pl.ds(j * _L, _L)] = acc
                        acc = cvec[0] * r_v[jq, pl.ds(0, _L)]
                        for gg in range(1, _G):
                            acc = acc + cvec[gg] * r_v[jq, pl.ds(gg * _L, _L)]
                        nacc[pl.ds(j * _L, _L)] = acc

                @pl.when(g < _RPW // 2 - 1)
                def _():
                    issue_gathers(r + 2, q)

                @pl.when(g >= 1)
                def _():
                    pltpu.make_async_copy(pos_sb[q], pos_hbm.at[b], osems[q]).wait()
                    pltpu.make_async_copy(neg_sb[q], neg_hbm.at[b], osems[q]).wait()

                @plsc.parallel_loop(0, _NP // _L, 1, unroll=2)
                def _(chunk):
                    rows = (chunk * _L + lane) * _L
                    tp = plsc.load_gather(pacc, [rows])
                    tn = plsc.load_gather(nacc, [rows])
                    for l in range(1, _L):
                        tp = tp + plsc.load_gather(pacc, [rows + l])
                        tn = tn + plsc.load_gather(nacc, [rows + l])
                    pos_sb[q][pl.ds(chunk * _L, _L)] = tp
                    neg_sb[q][pl.ds(chunk * _L, _L)] = tn

                pltpu.async_copy(pos_sb[q], pos_hbm.at[b], osems[q])
                pltpu.async_copy(neg_sb[q], neg_hbm.at[b], osems[q])
            return carry

        lax.fori_loop(0, _RPW // 2, pair_body, 0)

        for q in (0, 1):
            pltpu.make_async_copy(pos_sb[q], pos_hbm.at[base], osems[q]).wait()
            pltpu.make_async_copy(neg_sb[q], neg_hbm.at[base], osems[q]).wait()

    return k(seq, W, rnd)


def _loss_tc(pos, neg):
    def body(pos_ref, neg_ref, out_ref):
        p = pos_ref[...]
        n = neg_ref[...]
        lp = jnp.log(jax.nn.sigmoid(p) + 1e-08)
        ln = jnp.log(1.0 - jax.nn.sigmoid(n) + 1e-08)
        inv = 1.0 / float(_B * _NP)
        out_ref[0, 0] = -(jnp.sum(jnp.sum(lp, axis=1)) * inv
                          + jnp.sum(jnp.sum(ln, axis=1)) * inv)

    return pl.pallas_call(
        body,
        out_shape=jax.ShapeDtypeStruct((1, 1), jnp.float32),
        out_specs=pl.BlockSpec(memory_space=pltpu.SMEM),
    )(pos, neg)


_RND_CACHE = {}


def _rnd_ids():
    if "v" not in _RND_CACHE:
        with jax.ensure_compile_time_eval():
            r = jax.random.randint(jax.random.key(1), (_B, _NC, _K),
                                   0, _NUM_NODES)
            _RND_CACHE["v"] = np.asarray(r).reshape(_B, 2, _HALF).astype(np.int32)
    return _RND_CACHE["v"]


def kernel(seq, W):
    seq = seq.astype(jnp.int32)
    pos, neg = _sc_scores(seq, W, _rnd_ids())
    return _loss_tc(pos, neg)[0, 0]


# speedup vs baseline: 1.0731x; 1.0026x over previous
import functools

import jax
import jax.numpy as jnp
import numpy as np
from jax import lax
from jax.experimental import pallas as pl
from jax.experimental.pallas import tpu as pltpu
from jax.experimental.pallas import tpu_sc as plsc

_NUM_NODES = 100000
_D = 128
_CW = 2
_B = 4096
_S = 50
_NC = _S - 2 * _CW - 2
_K = 2 * _CW
_NP = _NC * _K
_L = 16
_G = _D // _L
_NW = 32
_RPW = _B // _NW
_HALF = _NP // 2
_OFFS = (0, 1, 3, 4)


def _sc_scores(seq, W, rnd):
    mesh = plsc.VectorSubcoreMesh(core_axis_name="c", subcore_axis_name="s")

    @functools.partial(
        pl.kernel,
        out_type=jax.ShapeDtypeStruct((_B, 2 * _NP), jnp.float32),
        mesh=mesh,
        compiler_params=pltpu.CompilerParams(needs_layout_passes=False),
        scratch_types=[
            pltpu.VMEM((_RPW, _S), jnp.int32),
            pltpu.VMEM((_RPW, 2, _HALF), jnp.int32),
            pltpu.VMEM((2 * _S, _D), jnp.float32),
            pltpu.VMEM((2 * _NP, _D), jnp.float32),
            pltpu.VMEM((2 * _NP * _L,), jnp.float32),
            pltpu.VMEM((2 * _NP,), jnp.float32),
            pltpu.VMEM((2 * _NP,), jnp.float32),
            pltpu.SemaphoreType.DMA,
            pltpu.SemaphoreType.DMA,
            pltpu.SemaphoreType.DMA,
            pltpu.SemaphoreType.DMA,
        ],
    )
    def k(seq_hbm, w_hbm, rnd_hbm, sco_hbm,
          seq_v, rnd_v, e_v, r_v, pacc, sco_s0, sco_s1,
          gsem0, gsem1, osem0, osem1):
        sco_sb = (sco_s0, sco_s1)
        wid = lax.axis_index("s") * 2 + lax.axis_index("c")
        base = wid * _RPW
        pltpu.sync_copy(seq_hbm.at[pl.ds(base, _RPW)], seq_v)
        pltpu.sync_copy(rnd_hbm.at[pl.ds(base, _RPW)], rnd_v)
        lane = lax.iota(jnp.int32, _L)
        gsems = (gsem0, gsem1)
        osems = (osem0, osem1)

        def gather_copies(r, q, sem):
            return (
                pltpu.make_async_copy(w_hbm.at[seq_v.at[r]],
                                      e_v.at[pl.ds(q * _S, _S)], sem),
                pltpu.make_async_copy(w_hbm.at[rnd_v.at[r, 0]],
                                      r_v.at[pl.ds(q * _NP, _HALF)], sem),
                pltpu.make_async_copy(w_hbm.at[rnd_v.at[r, 1]],
                                      r_v.at[pl.ds(q * _NP + _HALF, _HALF)], sem),
            )

        def issue_gathers(r, q):
            for c in gather_copies(r, q, gsems[q]):
                c.start()

        def drain_gathers(r, q):
            for c in gather_copies(r, q, gsems[q]):
                c.wait()

        issue_gathers(0, 0)
        issue_gathers(1, 1)

        def pair_body(g, carry):
            for q in (0, 1):
                r = 2 * g + q
                b = base + r
                drain_gathers(r, q)

                @plsc.parallel_loop(0, _NC, 1, unroll=2)
                def _(i):
                    m = i + _CW + q * _S
                    cvec = [e_v[m, pl.ds(gg * _L, _L)] for gg in range(_G)]
                    for kk, off in enumerate(_OFFS):
                        p = i + off + q * _S
                        j = i * _K + kk
                        jq = j + q * _NP
                        acc = cvec[0] * e_v[p, pl.ds(0, _L)]
                        for gg in range(1, _G):
                            acc = acc + cvec[gg] * e_v[p, pl.ds(gg * _L, _L)]
                        pacc[pl.ds(j * _L, _L)] = acc
                        acc = cvec[0] * r_v[jq, pl.ds(0, _L)]
                        for gg in range(1, _G):
                            acc = acc + cvec[gg] * r_v[jq, pl.ds(gg * _L, _L)]
                        pacc[pl.ds((_NP + j) * _L, _L)] = acc

                @pl.when(g < _RPW // 2 - 1)
                def _():
                    issue_gathers(r + 2, q)

                @pl.when(g >= 1)
                def _():
                    pltpu.make_async_copy(sco_sb[q], sco_hbm.at[b], osems[q]).wait()

                @plsc.parallel_loop(0, 2 * _NP // _L, 1, unroll=2)
                def _(chunk):
                    rows = (chunk * _L + lane) * _L
                    tp = plsc.load_gather(pacc, [rows])
                    for l in range(1, _L):
                        tp = tp + plsc.load_gather(pacc, [rows + l])
                    sco_sb[q][pl.ds(chunk * _L, _L)] = tp

                pltpu.async_copy(sco_sb[q], sco_hbm.at[b], osems[q])
            return carry

        lax.fori_loop(0, _RPW // 2, pair_body, 0)

        for q in (0, 1):
            pltpu.make_async_copy(sco_sb[q], sco_hbm.at[base], osems[q]).wait()

    return k(seq, W, rnd)


def _loss_tc(scores):
    def body(sco_ref, out_ref):
        p = sco_ref[:, :_NP]
        n = sco_ref[:, _NP:]
        lp = jnp.log(jax.nn.sigmoid(p) + 1e-08)
        ln = jnp.log(1.0 - jax.nn.sigmoid(n) + 1e-08)
        inv = 1.0 / float(_B * _NP)
        out_ref[0, 0] = -(jnp.sum(jnp.sum(lp, axis=1)) * inv
                          + jnp.sum(jnp.sum(ln, axis=1)) * inv)

    return pl.pallas_call(
        body,
        out_shape=jax.ShapeDtypeStruct((1, 1), jnp.float32),
        out_specs=pl.BlockSpec(memory_space=pltpu.SMEM),
    )(scores)


_RND_CACHE = {}


def _rnd_ids():
    if "v" not in _RND_CACHE:
        with jax.ensure_compile_time_eval():
            r = jax.random.randint(jax.random.key(1), (_B, _NC, _K),
                                   0, _NUM_NODES)
            _RND_CACHE["v"] = np.asarray(r).reshape(_B, 2, _HALF).astype(np.int32)
    return _RND_CACHE["v"]


def kernel(seq, W):
    seq = seq.astype(jnp.int32)
    scores = _sc_scores(seq, W, _rnd_ids())
    return _loss_tc(scores)[0, 0]
